# baseline (speedup 1.0000x reference)
import functools

import jax
import jax.numpy as jnp
from jax import lax
from jax.experimental import pallas as pl
from jax.experimental.pallas import tpu as pltpu
from jax.experimental.pallas import tpu_sc as plsc

N = 10000
E = 320000
D = 128
NUM_CENTROID = 100
NUM_CLASS = 10
NUM_GRAPHS = 128

NC = 2
NS = 16
NW = NC * NS
ACC_PAD = 10240
ROWS_PER_TILE = ACC_PAD // NS
ECHUNK = 128
NCHUNK = 80
E_PAD = NW * NCHUNK * ECHUNK
DEG_PAD = 10240
DEG_PER_TILE = DEG_PAD // NS

_HIGH = jax.lax.Precision.HIGHEST


def _mesh():
    return plsc.VectorSubcoreMesh(core_axis_name="c", subcore_axis_name="s")


def _deg_body(dst_hbm, deg_hbm, acc_sh, dstall, ones_v, zbuf):
    c = lax.axis_index("c")
    s = lax.axis_index("s")
    w = c * NS + s
    one16 = jnp.full((16,), 1.0, dtype=jnp.float32)
    zero16 = jnp.zeros((16,), dtype=jnp.float32)

    def fill_ones(k, _):
        ones_v[pl.ds(k * 16, 16)] = one16
        return 0

    lax.fori_loop(0, ECHUNK // 16, fill_ones, 0)

    def fill_zero(k, _):
        zbuf[pl.ds(k * 16, 16)] = zero16
        return 0

    lax.fori_loop(0, DEG_PER_TILE // 16, fill_zero, 0)
    pltpu.sync_copy(zbuf, acc_sh.at[pl.ds(s * DEG_PER_TILE, DEG_PER_TILE)])
    pltpu.sync_copy(dst_hbm.at[pl.ds(w * NCHUNK, NCHUNK)], dstall)
    plsc.subcore_barrier()

    def step(it, _):
        pltpu.sync_copy(ones_v, acc_sh.at[dstall.at[it]], add=True)
        return 0

    lax.fori_loop(0, NCHUNK, step, 0)
    plsc.subcore_barrier()
    pltpu.sync_copy(
        acc_sh.at[pl.ds(s * DEG_PER_TILE, DEG_PER_TILE)],
        deg_hbm.at[pl.ds(c * DEG_PAD + s * DEG_PER_TILE, DEG_PER_TILE)],
    )


def _sc_degree(dst2d):
    kern = pl.kernel(
        _deg_body,
        out_type=jax.ShapeDtypeStruct((NC * DEG_PAD,), jnp.float32),
        mesh=_mesh(),
        scratch_types=[
            pltpu.VMEM_SHARED((DEG_PAD,), jnp.float32),
            pltpu.VMEM((NCHUNK, ECHUNK), jnp.int32),
            pltpu.VMEM((ECHUNK,), jnp.float32),
            pltpu.VMEM((DEG_PER_TILE,), jnp.float32),
        ],
    )
    return kern(dst2d)


_PHCH = NCHUNK // 2


def _agg_body(g_hbm, src_hbm, dst_hbm, out_hbm, acc_sh, srcall, dstall,
              rows_a, rows_b, sem_a, sem_b):
    c = lax.axis_index("c")
    s = lax.axis_index("s")
    w = c * NS + s
    zero16 = jnp.zeros((16,), dtype=jnp.float32)

    def zrow(r, _):
        for j in range(D // 16):
            rows_a[r, pl.ds(j * 16, 16)] = zero16
        return 0

    lax.fori_loop(0, ECHUNK, zrow, 0)
    for rr in range(ROWS_PER_TILE // ECHUNK):
        pltpu.sync_copy(rows_a,
                        acc_sh.at[pl.ds(s * ROWS_PER_TILE + rr * ECHUNK, ECHUNK)])
    plsc.subcore_barrier()

    def step(k, _):
        it0 = 2 * k
        it1 = 2 * k + 1
        pltpu.async_copy(g_hbm.at[srcall.at[it1]], rows_b, sem_b)
        pltpu.make_async_copy(g_hbm.at[srcall.at[it0]], rows_a, sem_a).wait()
        pltpu.sync_copy(rows_a, acc_sh.at[dstall.at[it0]], add=True)

        @pl.when(k < _PHCH // 2 - 1)
        def _():
            pltpu.async_copy(g_hbm.at[srcall.at[it1 + 1]], rows_a, sem_a)

        pltpu.make_async_copy(g_hbm.at[srcall.at[it1]], rows_b, sem_b).wait()
        pltpu.sync_copy(rows_b, acc_sh.at[dstall.at[it1]], add=True)
        return 0

    for ph in range(NCHUNK // _PHCH):
        pltpu.sync_copy(src_hbm.at[pl.ds(w * NCHUNK + ph * _PHCH, _PHCH)], srcall)
        pltpu.sync_copy(dst_hbm.at[pl.ds(w * NCHUNK + ph * _PHCH, _PHCH)], dstall)
        pltpu.async_copy(g_hbm.at[srcall.at[0]], rows_a, sem_a)
        lax.fori_loop(0, _PHCH // 2, step, 0)

    plsc.subcore_barrier()
    pltpu.sync_copy(
        acc_sh.at[pl.ds(s * ROWS_PER_TILE, ROWS_PER_TILE)],
        out_hbm.at[pl.ds(c * ACC_PAD + s * ROWS_PER_TILE, ROWS_PER_TILE)],
    )


def _sc_aggregate(g, src2d, dst2d):
    kern = pl.kernel(
        _agg_body,
        out_type=jax.ShapeDtypeStruct((NC * ACC_PAD, D), jnp.float32),
        mesh=_mesh(),
        scratch_types=[
            pltpu.VMEM_SHARED((ACC_PAD, D), jnp.float32),
            pltpu.VMEM((_PHCH, ECHUNK), jnp.int32),
            pltpu.VMEM((_PHCH, ECHUNK), jnp.int32),
            pltpu.VMEM((ECHUNK, D), jnp.float32),
            pltpu.VMEM((ECHUNK, D), jnp.float32),
            pltpu.SemaphoreType.DMA,
            pltpu.SemaphoreType.DMA,
        ],
    )
    return kern(g, src2d, dst2d)


_RB = 1000
_GRID = N // _RB


def _k2_body(x_ref, degp_ref, we_ref, w0_ref, g1_ref, dinv_ref):
    deg = degp_ref[0, :, 0] + degp_ref[1, :, 0] + 1.0
    dv = lax.rsqrt(deg)
    t = lax.dot_general(x_ref[...], we_ref[...], (((1,), (0,)), ((), ())),
                        precision=_HIGH, preferred_element_type=jnp.float32)
    t = lax.dot_general(t, w0_ref[...], (((1,), (0,)), ((), ())),
                        precision=_HIGH, preferred_element_type=jnp.float32)
    g1_ref[...] = dv[:, None] * t
    dinv_ref[...] = dv[:, None]


def _tc_embed_scale(x, degp, W_embed, W0):
    return pl.pallas_call(
        _k2_body,
        grid=(_GRID,),
        in_specs=[
            pl.BlockSpec((_RB, D), lambda i: (i, 0)),
            pl.BlockSpec((NC, _RB, 1), lambda i: (0, i, 0)),
            pl.BlockSpec((D, D), lambda i: (0, 0)),
            pl.BlockSpec((D, D), lambda i: (0, 0)),
        ],
        out_specs=[
            pl.BlockSpec((_RB, D), lambda i: (i, 0)),
            pl.BlockSpec((_RB, 1), lambda i: (i, 0)),
        ],
        out_shape=[
            jax.ShapeDtypeStruct((N, D), jnp.float32),
            jax.ShapeDtypeStruct((N, 1), jnp.float32),
        ],
    )(x, degp, W_embed, W0)


def _k4_body(acc_ref, g1_ref, dinv_ref, w1_ref, b0_ref, g2_ref):
    dv = dinv_ref[...]
    a = acc_ref[0] + acc_ref[1] + g1_ref[...]
    h1 = jnp.maximum(dv * a + b0_ref[...], 0.0)
    t = lax.dot_general(h1, w1_ref[...], (((1,), (0,)), ((), ())),
                        precision=_HIGH, preferred_element_type=jnp.float32)
    g2_ref[...] = dv * t


def _tc_layer2_prep(acc1, g1, dinv, W1, b0):
    return pl.pallas_call(
        _k4_body,
        grid=(_GRID,),
        in_specs=[
            pl.BlockSpec((NC, _RB, D), lambda i: (0, i, 0)),
            pl.BlockSpec((_RB, D), lambda i: (i, 0)),
            pl.BlockSpec((_RB, 1), lambda i: (i, 0)),
            pl.BlockSpec((D, D), lambda i: (0, 0)),
            pl.BlockSpec((1, D), lambda i: (0, 0)),
        ],
        out_specs=pl.BlockSpec((_RB, D), lambda i: (i, 0)),
        out_shape=jax.ShapeDtypeStruct((N, D), jnp.float32),
    )(acc1, g1, dinv, W1, b0)


def _k6_body(acc_ref, g2_ref, dinv_ref, batch_ref, cent_ref, b1_ref,
             wout_ref, bout_ref, out_ref, pacc):
    i = pl.program_id(0)

    @pl.when(i == 0)
    def _init():
        pacc[...] = jnp.zeros_like(pacc)

    dv = dinv_ref[...]
    a = acc_ref[0] + acc_ref[1] + g2_ref[...]
    h2 = jnp.maximum(dv * a + b1_ref[...], 0.0)

    cent = cent_ref[...]
    csq = jnp.sum(cent * cent, axis=1)
    hc = lax.dot_general(h2, cent, (((1,), (1,)), ((), ())),
                         precision=_HIGH, preferred_element_type=jnp.float32)
    sq = jnp.sum(h2 * h2, axis=1, keepdims=True) + csq[None, :] - 2.0 * hc
    dist = jnp.sqrt(jnp.maximum(sq, 1e-8))
    dist1 = jnp.concatenate(
        [dist, jnp.ones((dist.shape[0], 1), jnp.float32)], axis=1)

    gids = lax.broadcasted_iota(jnp.int32, (_RB, NUM_GRAPHS), 1)
    oh = (batch_ref[...] == gids).astype(jnp.float32)
    pacc[...] += lax.dot_general(oh, dist1, (((0,), (0,)), ((), ())),
                                 precision=_HIGH,
                                 preferred_element_type=jnp.float32)

    @pl.when(i == _GRID - 1)
    def _final():
        p = pacc[...]
        pooled = p[:, :NUM_CENTROID] / jnp.maximum(p[:, NUM_CENTROID:], 1.0)
        out_ref[...] = lax.dot_general(
            pooled, wout_ref[...], (((1,), (0,)), ((), ())),
            precision=_HIGH, preferred_element_type=jnp.float32) + bout_ref[...]


def _tc_pool_out(acc2, g2, dinv, batch2d, centroids, b1, W_out, b_out):
    return pl.pallas_call(
        _k6_body,
        grid=(_GRID,),
        in_specs=[
            pl.BlockSpec((NC, _RB, D), lambda i: (0, i, 0)),
            pl.BlockSpec((_RB, D), lambda i: (i, 0)),
            pl.BlockSpec((_RB, 1), lambda i: (i, 0)),
            pl.BlockSpec((_RB, 1), lambda i: (i, 0)),
            pl.BlockSpec((NUM_CENTROID, D), lambda i: (0, 0)),
            pl.BlockSpec((1, D), lambda i: (0, 0)),
            pl.BlockSpec((NUM_CENTROID, NUM_CLASS), lambda i: (0, 0)),
            pl.BlockSpec((1, NUM_CLASS), lambda i: (0, 0)),
        ],
        out_specs=pl.BlockSpec((NUM_GRAPHS, NUM_CLASS), lambda i: (0, 0)),
        out_shape=jax.ShapeDtypeStruct((NUM_GRAPHS, NUM_CLASS), jnp.float32),
        scratch_shapes=[pltpu.VMEM((NUM_GRAPHS, NUM_CENTROID + 1), jnp.float32)],
    )(acc2, g2, dinv, batch2d, centroids, b1, W_out, b_out)


def kernel(x, edge_index, batch, W_embed, W_gcn, b_gcn, centroids, W_out, b_out):
    npad = E_PAD - E
    src2d = jnp.concatenate(
        [edge_index[0], jnp.zeros((npad,), jnp.int32)]).reshape(-1, ECHUNK)
    dst2d = jnp.concatenate(
        [edge_index[1], jnp.full((npad,), ACC_PAD - 1, jnp.int32)]
    ).reshape(-1, ECHUNK)
    batch2d = batch.reshape(N, 1)
    b0 = b_gcn[0].reshape(1, D)
    b1 = b_gcn[1].reshape(1, D)
    bout = b_out.reshape(1, NUM_CLASS)

    degp = _sc_degree(dst2d).reshape(NC, DEG_PAD, 1)
    g1, dinv = _tc_embed_scale(x, degp, W_embed, W_gcn[0])
    acc1 = _sc_aggregate(g1, src2d, dst2d).reshape(NC, ACC_PAD, D)
    g2 = _tc_layer2_prep(acc1, g1, dinv, W_gcn[1], b0)
    acc2 = _sc_aggregate(g2, src2d, dst2d).reshape(NC, ACC_PAD, D)
    return _tc_pool_out(acc2, g2, dinv, batch2d, centroids, b1, W_out, bout)


# speedup vs baseline: 3.0049x; 3.0049x over previous
import functools

import jax
import jax.numpy as jnp
from jax import lax
from jax.experimental import pallas as pl
from jax.experimental.pallas import tpu as pltpu
from jax.experimental.pallas import tpu_sc as plsc

N = 10000
E = 320000
D = 128
NUM_CENTROID = 100
NUM_CLASS = 10
NUM_GRAPHS = 128

NC = 2
NS = 16
NW = NC * NS
ACC_PAD = 10240
ROWS_PER_TILE = ACC_PAD // NS
ECHUNK = 128
NCHUNK = 80
E_PAD = NW * NCHUNK * ECHUNK
DEG_PAD = 10240
DEG_PER_TILE = DEG_PAD // NS

_HIGH = jax.lax.Precision.HIGHEST


def _mesh():
    return plsc.VectorSubcoreMesh(core_axis_name="c", subcore_axis_name="s")


def _deg_body(dst_hbm, deg_hbm, acc_sh, dstall, ones_v, zbuf):
    c = lax.axis_index("c")
    s = lax.axis_index("s")
    w = c * NS + s
    one16 = jnp.full((16,), 1.0, dtype=jnp.float32)
    zero16 = jnp.zeros((16,), dtype=jnp.float32)

    def fill_ones(k, _):
        ones_v[pl.ds(k * 16, 16)] = one16
        return 0

    lax.fori_loop(0, ECHUNK // 16, fill_ones, 0)

    def fill_zero(k, _):
        zbuf[pl.ds(k * 16, 16)] = zero16
        return 0

    lax.fori_loop(0, DEG_PER_TILE // 16, fill_zero, 0)
    pltpu.sync_copy(zbuf, acc_sh.at[pl.ds(s * DEG_PER_TILE, DEG_PER_TILE)])
    pltpu.sync_copy(dst_hbm.at[pl.ds(w * NCHUNK, NCHUNK)], dstall)
    plsc.subcore_barrier()

    def step(it, _):
        pltpu.sync_copy(ones_v, acc_sh.at[dstall.at[it]], add=True)
        return 0

    lax.fori_loop(0, NCHUNK, step, 0)
    plsc.subcore_barrier()
    pltpu.sync_copy(
        acc_sh.at[pl.ds(s * DEG_PER_TILE, DEG_PER_TILE)],
        deg_hbm.at[pl.ds(c * DEG_PAD + s * DEG_PER_TILE, DEG_PER_TILE)],
    )


def _sc_degree(dst2d):
    kern = pl.kernel(
        _deg_body,
        out_type=jax.ShapeDtypeStruct((NC * DEG_PAD,), jnp.float32),
        mesh=_mesh(),
        scratch_types=[
            pltpu.VMEM_SHARED((DEG_PAD,), jnp.float32),
            pltpu.VMEM((NCHUNK, ECHUNK), jnp.int32),
            pltpu.VMEM((ECHUNK,), jnp.float32),
            pltpu.VMEM((DEG_PER_TILE,), jnp.float32),
        ],
    )
    return kern(dst2d)


_PHCH = NCHUNK // 2


def _agg_body(g_hbm, src_hbm, dst_hbm, out_hbm, acc_sh, srcall, dstall,
              rows_a, rows_b, sem_a, sem_b):
    c = lax.axis_index("c")
    s = lax.axis_index("s")
    w = c * NS + s
    zero16 = jnp.zeros((16,), dtype=jnp.float32)

    def zrow(r, _):
        for j in range(D // 16):
            rows_a[r, pl.ds(j * 16, 16)] = zero16
        return 0

    lax.fori_loop(0, ECHUNK, zrow, 0)
    for rr in range(ROWS_PER_TILE // ECHUNK):
        pltpu.sync_copy(rows_a,
                        acc_sh.at[pl.ds(s * ROWS_PER_TILE + rr * ECHUNK, ECHUNK)])
    plsc.subcore_barrier()

    def step(k, _):
        it0 = 2 * k
        it1 = 2 * k + 1
        pltpu.async_copy(g_hbm.at[srcall.at[it1]], rows_b, sem_b)
        pltpu.make_async_copy(g_hbm.at[srcall.at[it0]], rows_a, sem_a).wait()
        pltpu.sync_copy(rows_a, acc_sh.at[dstall.at[it0]], add=True)

        @pl.when(k < _PHCH // 2 - 1)
        def _():
            pltpu.async_copy(g_hbm.at[srcall.at[it1 + 1]], rows_a, sem_a)

        pltpu.make_async_copy(g_hbm.at[srcall.at[it1]], rows_b, sem_b).wait()
        pltpu.sync_copy(rows_b, acc_sh.at[dstall.at[it1]], add=True)
        return 0

    for ph in range(NCHUNK // _PHCH):
        pltpu.sync_copy(src_hbm.at[pl.ds(w * NCHUNK + ph * _PHCH, _PHCH)], srcall)
        pltpu.sync_copy(dst_hbm.at[pl.ds(w * NCHUNK + ph * _PHCH, _PHCH)], dstall)
        pltpu.async_copy(g_hbm.at[srcall.at[0]], rows_a, sem_a)
        lax.fori_loop(0, _PHCH // 2, step, 0)

    plsc.subcore_barrier()
    pltpu.sync_copy(
        acc_sh.at[pl.ds(s * ROWS_PER_TILE, ROWS_PER_TILE)],
        out_hbm.at[pl.ds(c * ACC_PAD + s * ROWS_PER_TILE, ROWS_PER_TILE)],
    )


def _sc_aggregate(g, src2d, dst2d):
    kern = pl.kernel(
        _agg_body,
        out_type=jax.ShapeDtypeStruct((NC * ACC_PAD, D), jnp.float32),
        mesh=_mesh(),
        scratch_types=[
            pltpu.VMEM_SHARED((ACC_PAD, D), jnp.float32),
            pltpu.VMEM((_PHCH, ECHUNK), jnp.int32),
            pltpu.VMEM((_PHCH, ECHUNK), jnp.int32),
            pltpu.VMEM((ECHUNK, D), jnp.float32),
            pltpu.VMEM((ECHUNK, D), jnp.float32),
            pltpu.SemaphoreType.DMA,
            pltpu.SemaphoreType.DMA,
        ],
    )
    return kern(g, src2d, dst2d)


_RB = 1000
_GRID = N // _RB


def _k2_body(x_ref, degp_ref, we_ref, w0_ref, g1_ref, dinv_ref):
    deg = degp_ref[0, :, 0] + degp_ref[1, :, 0] + 1.0
    dv = lax.rsqrt(deg)
    t = lax.dot_general(x_ref[...], we_ref[...], (((1,), (0,)), ((), ())),
                        precision=_HIGH, preferred_element_type=jnp.float32)
    t = lax.dot_general(t, w0_ref[...], (((1,), (0,)), ((), ())),
                        precision=_HIGH, preferred_element_type=jnp.float32)
    g1_ref[...] = dv[:, None] * t
    dinv_ref[...] = dv[:, None]


def _tc_embed_scale(x, degp, W_embed, W0):
    return pl.pallas_call(
        _k2_body,
        grid=(_GRID,),
        in_specs=[
            pl.BlockSpec((_RB, D), lambda i: (i, 0)),
            pl.BlockSpec((NC, _RB, 1), lambda i: (0, i, 0)),
            pl.BlockSpec((D, D), lambda i: (0, 0)),
            pl.BlockSpec((D, D), lambda i: (0, 0)),
        ],
        out_specs=[
            pl.BlockSpec((_RB, D), lambda i: (i, 0)),
            pl.BlockSpec((_RB, 1), lambda i: (i, 0)),
        ],
        out_shape=[
            jax.ShapeDtypeStruct((N, D), jnp.float32),
            jax.ShapeDtypeStruct((N, 1), jnp.float32),
        ],
    )(x, degp, W_embed, W0)


def _k4_body(acc_ref, g1_ref, dinv_ref, w1_ref, b0_ref, g2_ref):
    dv = dinv_ref[...]
    a = acc_ref[0] + acc_ref[1] + g1_ref[...]
    h1 = jnp.maximum(dv * a + b0_ref[...], 0.0)
    t = lax.dot_general(h1, w1_ref[...], (((1,), (0,)), ((), ())),
                        precision=_HIGH, preferred_element_type=jnp.float32)
    g2_ref[...] = dv * t


def _tc_layer2_prep(acc1, g1, dinv, W1, b0):
    return pl.pallas_call(
        _k4_body,
        grid=(_GRID,),
        in_specs=[
            pl.BlockSpec((NC, _RB, D), lambda i: (0, i, 0)),
            pl.BlockSpec((_RB, D), lambda i: (i, 0)),
            pl.BlockSpec((_RB, 1), lambda i: (i, 0)),
            pl.BlockSpec((D, D), lambda i: (0, 0)),
            pl.BlockSpec((1, D), lambda i: (0, 0)),
        ],
        out_specs=pl.BlockSpec((_RB, D), lambda i: (i, 0)),
        out_shape=jax.ShapeDtypeStruct((N, D), jnp.float32),
    )(acc1, g1, dinv, W1, b0)


def _k6_body(acc_ref, g2_ref, dinv_ref, batch_ref, cent_ref, b1_ref,
             wout_ref, bout_ref, out_ref, pacc):
    i = pl.program_id(0)

    @pl.when(i == 0)
    def _init():
        pacc[...] = jnp.zeros_like(pacc)

    dv = dinv_ref[...]
    a = acc_ref[0] + acc_ref[1] + g2_ref[...]
    h2 = jnp.maximum(dv * a + b1_ref[...], 0.0)

    cent = cent_ref[...]
    csq = jnp.sum(cent * cent, axis=1)
    hc = lax.dot_general(h2, cent, (((1,), (1,)), ((), ())),
                         precision=_HIGH, preferred_element_type=jnp.float32)
    sq = jnp.sum(h2 * h2, axis=1, keepdims=True) + csq[None, :] - 2.0 * hc
    dist = jnp.sqrt(jnp.maximum(sq, 1e-8))
    dist1 = jnp.concatenate(
        [dist, jnp.ones((dist.shape[0], 1), jnp.float32)], axis=1)

    gids = lax.broadcasted_iota(jnp.int32, (_RB, NUM_GRAPHS), 1)
    oh = (batch_ref[...] == gids).astype(jnp.float32)
    pacc[...] += lax.dot_general(oh, dist1, (((0,), (0,)), ((), ())),
                                 precision=_HIGH,
                                 preferred_element_type=jnp.float32)

    @pl.when(i == _GRID - 1)
    def _final():
        p = pacc[...]
        pooled = p[:, :NUM_CENTROID] / jnp.maximum(p[:, NUM_CENTROID:], 1.0)
        out_ref[...] = lax.dot_general(
            pooled, wout_ref[...], (((1,), (0,)), ((), ())),
            precision=_HIGH, preferred_element_type=jnp.float32) + bout_ref[...]


def _tc_pool_out(acc2, g2, dinv, batch2d, centroids, b1, W_out, b_out):
    return pl.pallas_call(
        _k6_body,
        grid=(_GRID,),
        in_specs=[
            pl.BlockSpec((NC, _RB, D), lambda i: (0, i, 0)),
            pl.BlockSpec((_RB, D), lambda i: (i, 0)),
            pl.BlockSpec((_RB, 1), lambda i: (i, 0)),
            pl.BlockSpec((_RB, 1), lambda i: (i, 0)),
            pl.BlockSpec((NUM_CENTROID, D), lambda i: (0, 0)),
            pl.BlockSpec((1, D), lambda i: (0, 0)),
            pl.BlockSpec((NUM_CENTROID, NUM_CLASS), lambda i: (0, 0)),
            pl.BlockSpec((1, NUM_CLASS), lambda i: (0, 0)),
        ],
        out_specs=pl.BlockSpec((NUM_GRAPHS, NUM_CLASS), lambda i: (0, 0)),
        out_shape=jax.ShapeDtypeStruct((NUM_GRAPHS, NUM_CLASS), jnp.float32),
        scratch_shapes=[pltpu.VMEM((NUM_GRAPHS, NUM_CENTROID + 1), jnp.float32)],
    )(acc2, g2, dinv, batch2d, centroids, b1, W_out, b_out)


def kernel(x, edge_index, batch, W_embed, W_gcn, b_gcn, centroids, W_out, b_out):
    npad = E_PAD - E
    pad_dst = N + jax.lax.iota(jnp.int32, npad) % (ACC_PAD - N)
    src2d = jnp.concatenate(
        [edge_index[0], jax.lax.iota(jnp.int32, npad) % N]).reshape(-1, ECHUNK)
    dst2d = jnp.concatenate([edge_index[1], pad_dst]).reshape(-1, ECHUNK)
    batch2d = batch.reshape(N, 1)
    b0 = b_gcn[0].reshape(1, D)
    b1 = b_gcn[1].reshape(1, D)
    bout = b_out.reshape(1, NUM_CLASS)

    degp = _sc_degree(dst2d).reshape(NC, DEG_PAD, 1)
    g1, dinv = _tc_embed_scale(x, degp, W_embed, W_gcn[0])
    acc1 = _sc_aggregate(g1, src2d, dst2d).reshape(NC, ACC_PAD, D)
    g2 = _tc_layer2_prep(acc1, g1, dinv, W_gcn[1], b0)
    acc2 = _sc_aggregate(g2, src2d, dst2d).reshape(NC, ACC_PAD, D)
    return _tc_pool_out(acc2, g2, dinv, batch2d, centroids, b1, W_out, bout)


# speedup vs baseline: 3.4039x; 1.1328x over previous
import functools

import jax
import jax.numpy as jnp
import numpy as np
from jax import lax
from jax.experimental import pallas as pl
from jax.experimental.pallas import tpu as pltpu
from jax.experimental.pallas import tpu_sc as plsc

N = 10000
E = 320000
D = 128
NUM_CENTROID = 100
NUM_CLASS = 10
NUM_GRAPHS = 128

NC = 2
NS = 16
NW = NC * NS
ACC_PAD = 10240
ROWS_PER_TILE = ACC_PAD // NS
ECHUNK = 128
NCHUNK = 80
E_PAD = NW * NCHUNK * ECHUNK
DEG_PAD = 10240
DEG_PER_TILE = DEG_PAD // NS

_HIGH = jax.lax.Precision.HIGHEST


def _mesh():
    return plsc.VectorSubcoreMesh(core_axis_name="c", subcore_axis_name="s")


def _deg_body(dst_hbm, deg_hbm, acc_sh, dstall, ones_v, zbuf):
    c = lax.axis_index("c")
    s = lax.axis_index("s")
    w = c * NS + s
    one16 = jnp.full((16,), 1.0, dtype=jnp.float32)
    zero16 = jnp.zeros((16,), dtype=jnp.float32)

    def fill_ones(k, _):
        ones_v[pl.ds(k * 16, 16)] = one16
        return 0

    lax.fori_loop(0, ECHUNK // 16, fill_ones, 0)

    def fill_zero(k, _):
        zbuf[pl.ds(k * 16, 16)] = zero16
        return 0

    lax.fori_loop(0, DEG_PER_TILE // 16, fill_zero, 0)
    pltpu.sync_copy(zbuf, acc_sh.at[pl.ds(s * DEG_PER_TILE, DEG_PER_TILE)])
    pltpu.sync_copy(dst_hbm.at[pl.ds(w * NCHUNK, NCHUNK)], dstall)
    plsc.subcore_barrier()

    def step(it, _):
        pltpu.sync_copy(ones_v, acc_sh.at[dstall.at[it]], add=True)
        return 0

    lax.fori_loop(0, NCHUNK, step, 0)
    plsc.subcore_barrier()
    pltpu.sync_copy(
        acc_sh.at[pl.ds(s * DEG_PER_TILE, DEG_PER_TILE)],
        deg_hbm.at[pl.ds(c * DEG_PAD + s * DEG_PER_TILE, DEG_PER_TILE)],
    )


def _sc_degree(dst2d):
    kern = pl.kernel(
        _deg_body,
        out_type=jax.ShapeDtypeStruct((NC * DEG_PAD,), jnp.float32),
        mesh=_mesh(),
        scratch_types=[
            pltpu.VMEM_SHARED((DEG_PAD,), jnp.float32),
            pltpu.VMEM((NCHUNK, ECHUNK), jnp.int32),
            pltpu.VMEM((ECHUNK,), jnp.float32),
            pltpu.VMEM((DEG_PER_TILE,), jnp.float32),
        ],
    )
    return kern(dst2d)


_PHCH = NCHUNK // 2


def _agg_body(g_hbm, src_hbm, dst_hbm, out_hbm, acc_sh, srcall, dstall,
              rows_a, rows_b, sem_a, sem_b):
    c = lax.axis_index("c")
    s = lax.axis_index("s")
    w = c * NS + s
    zero16 = jnp.zeros((16,), dtype=jnp.float32)

    def zrow(r, _):
        for j in range(D // 16):
            rows_a[r, pl.ds(j * 16, 16)] = zero16
        return 0

    lax.fori_loop(0, ECHUNK, zrow, 0)
    for rr in range(ROWS_PER_TILE // ECHUNK):
        pltpu.sync_copy(rows_a,
                        acc_sh.at[pl.ds(s * ROWS_PER_TILE + rr * ECHUNK, ECHUNK)])
    plsc.subcore_barrier()

    def step(k, _):
        it0 = 2 * k
        it1 = 2 * k + 1
        pltpu.async_copy(g_hbm.at[srcall.at[it1]], rows_b, sem_b)
        pltpu.make_async_copy(g_hbm.at[srcall.at[it0]], rows_a, sem_a).wait()
        pltpu.sync_copy(rows_a, acc_sh.at[dstall.at[it0]], add=True)

        @pl.when(k < _PHCH // 2 - 1)
        def _():
            pltpu.async_copy(g_hbm.at[srcall.at[it1 + 1]], rows_a, sem_a)

        pltpu.make_async_copy(g_hbm.at[srcall.at[it1]], rows_b, sem_b).wait()
        pltpu.sync_copy(rows_b, acc_sh.at[dstall.at[it1]], add=True)
        return 0

    for ph in range(NCHUNK // _PHCH):
        pltpu.sync_copy(src_hbm.at[pl.ds(w * NCHUNK + ph * _PHCH, _PHCH)], srcall)
        pltpu.sync_copy(dst_hbm.at[pl.ds(w * NCHUNK + ph * _PHCH, _PHCH)], dstall)
        pltpu.async_copy(g_hbm.at[srcall.at[0]], rows_a, sem_a)
        lax.fori_loop(0, _PHCH // 2, step, 0)

    plsc.subcore_barrier()
    pltpu.sync_copy(
        acc_sh.at[pl.ds(s * ROWS_PER_TILE, ROWS_PER_TILE)],
        out_hbm.at[pl.ds(c * ACC_PAD + s * ROWS_PER_TILE, ROWS_PER_TILE)],
    )


def _sc_aggregate(g, src2d, dst2d):
    kern = pl.kernel(
        _agg_body,
        out_type=jax.ShapeDtypeStruct((NC * ACC_PAD, D), jnp.float32),
        mesh=_mesh(),
        scratch_types=[
            pltpu.VMEM_SHARED((ACC_PAD, D), jnp.float32),
            pltpu.VMEM((_PHCH, ECHUNK), jnp.int32),
            pltpu.VMEM((_PHCH, ECHUNK), jnp.int32),
            pltpu.VMEM((ECHUNK, D), jnp.float32),
            pltpu.VMEM((ECHUNK, D), jnp.float32),
            pltpu.SemaphoreType.DMA,
            pltpu.SemaphoreType.DMA,
        ],
    )
    return kern(g, src2d, dst2d)


_RB = 1000
_GRID = N // _RB
_DEF = jax.lax.Precision.DEFAULT


def _dinv_col(degp_blk):
    return lax.rsqrt(degp_blk[:, 0:1] + degp_blk[:, 1:2] + 1.0)


def _k2a_body(x_ref, we_ref, w0_ref, t_ref):
    t = lax.dot_general(x_ref[...], we_ref[...], (((1,), (0,)), ((), ())),
                        precision=_DEF, preferred_element_type=jnp.float32)
    t_ref[...] = lax.dot_general(t, w0_ref[...], (((1,), (0,)), ((), ())),
                                 precision=_DEF,
                                 preferred_element_type=jnp.float32)


def _tc_embed(x, W_embed, W0):
    return pl.pallas_call(
        _k2a_body,
        grid=(_GRID,),
        in_specs=[
            pl.BlockSpec((_RB, D), lambda i: (i, 0)),
            pl.BlockSpec((D, D), lambda i: (0, 0)),
            pl.BlockSpec((D, D), lambda i: (0, 0)),
        ],
        out_specs=pl.BlockSpec((_RB, D), lambda i: (i, 0)),
        out_shape=jax.ShapeDtypeStruct((N, D), jnp.float32),
    )(x, W_embed, W0)


def _k2b_body(t_ref, degp_ref, g1_ref):
    g1_ref[...] = _dinv_col(degp_ref[...]) * t_ref[...]


def _tc_scale(t, degp):
    return pl.pallas_call(
        _k2b_body,
        grid=(_GRID,),
        in_specs=[
            pl.BlockSpec((_RB, D), lambda i: (i, 0)),
            pl.BlockSpec((_RB, NC), lambda i: (i, 0)),
        ],
        out_specs=pl.BlockSpec((_RB, D), lambda i: (i, 0)),
        out_shape=jax.ShapeDtypeStruct((N, D), jnp.float32),
    )(t, degp)


def _k4_body(acc_ref, g1_ref, degp_ref, w1_ref, b0_ref, g2_ref):
    dv = _dinv_col(degp_ref[...])
    a = acc_ref[0] + acc_ref[1] + g1_ref[...]
    h1 = jnp.maximum(dv * a + b0_ref[...], 0.0)
    t = lax.dot_general(h1, w1_ref[...], (((1,), (0,)), ((), ())),
                        precision=_DEF, preferred_element_type=jnp.float32)
    g2_ref[...] = dv * t


def _tc_layer2_prep(acc1, g1, degp, W1, b0):
    return pl.pallas_call(
        _k4_body,
        grid=(_GRID,),
        in_specs=[
            pl.BlockSpec((NC, _RB, D), lambda i: (0, i, 0)),
            pl.BlockSpec((_RB, D), lambda i: (i, 0)),
            pl.BlockSpec((_RB, NC), lambda i: (i, 0)),
            pl.BlockSpec((D, D), lambda i: (0, 0)),
            pl.BlockSpec((1, D), lambda i: (0, 0)),
        ],
        out_specs=pl.BlockSpec((_RB, D), lambda i: (i, 0)),
        out_shape=jax.ShapeDtypeStruct((N, D), jnp.float32),
    )(acc1, g1, degp, W1, b0)


def _k6_body(acc_ref, g2_ref, degp_ref, batch_ref, cent_ref, b1_ref,
             wout_ref, bout_ref, out_ref, pacc):
    i = pl.program_id(0)

    @pl.when(i == 0)
    def _init():
        pacc[...] = jnp.zeros_like(pacc)

    dv = _dinv_col(degp_ref[...])
    a = acc_ref[0] + acc_ref[1] + g2_ref[...]
    h2 = jnp.maximum(dv * a + b1_ref[...], 0.0)

    cent = cent_ref[...]
    csq = jnp.sum(cent * cent, axis=1)
    hc = lax.dot_general(h2, cent, (((1,), (1,)), ((), ())),
                         precision=_DEF, preferred_element_type=jnp.float32)
    sq = jnp.sum(h2 * h2, axis=1, keepdims=True) + csq[None, :] - 2.0 * hc
    dist = jnp.sqrt(jnp.maximum(sq, 1e-8))
    dist1 = jnp.concatenate(
        [dist, jnp.ones((dist.shape[0], 1), jnp.float32)], axis=1)

    gids = lax.broadcasted_iota(jnp.int32, (NUM_GRAPHS, _RB), 0)
    bt = batch_ref[...].reshape(1, _RB)
    oht = (bt == gids).astype(jnp.float32)
    pacc[...] += lax.dot_general(oht, dist1, (((1,), (0,)), ((), ())),
                                 precision=_HIGH,
                                 preferred_element_type=jnp.float32)

    @pl.when(i == _GRID - 1)
    def _final():
        p = pacc[...]
        pooled = p[:, :NUM_CENTROID] / jnp.maximum(p[:, NUM_CENTROID:], 1.0)
        out_ref[...] = lax.dot_general(
            pooled, wout_ref[...], (((1,), (0,)), ((), ())),
            precision=_HIGH, preferred_element_type=jnp.float32) + bout_ref[...]


def _tc_pool_out(acc2, g2, degp, batch, centroids, b1, W_out, b_out):
    return pl.pallas_call(
        _k6_body,
        grid=(_GRID,),
        in_specs=[
            pl.BlockSpec((NC, _RB, D), lambda i: (0, i, 0)),
            pl.BlockSpec((_RB, D), lambda i: (i, 0)),
            pl.BlockSpec((_RB, NC), lambda i: (i, 0)),
            pl.BlockSpec((1, 1, _RB), lambda i: (i, 0, 0)),
            pl.BlockSpec((NUM_CENTROID, D), lambda i: (0, 0)),
            pl.BlockSpec((1, D), lambda i: (0, 0)),
            pl.BlockSpec((NUM_CENTROID, NUM_CLASS), lambda i: (0, 0)),
            pl.BlockSpec((1, NUM_CLASS), lambda i: (0, 0)),
        ],
        out_specs=pl.BlockSpec((NUM_GRAPHS, NUM_CLASS), lambda i: (0, 0)),
        out_shape=jax.ShapeDtypeStruct((NUM_GRAPHS, NUM_CLASS), jnp.float32),
        scratch_shapes=[pltpu.VMEM((NUM_GRAPHS, NUM_CENTROID + 1), jnp.float32)],
    )(acc2, g2, degp, batch, centroids, b1, W_out, b_out)


_NPAD = E_PAD - E
_PAD_SRC = np.arange(_NPAD, dtype=np.int32) % N
_PAD_DST = (N + np.arange(_NPAD, dtype=np.int32) % (ACC_PAD - N)).astype(np.int32)


def kernel(x, edge_index, batch, W_embed, W_gcn, b_gcn, centroids, W_out, b_out):
    src2d = jnp.concatenate([edge_index[0], jnp.asarray(_PAD_SRC)]).reshape(-1, ECHUNK)
    dst2d = jnp.concatenate([edge_index[1], jnp.asarray(_PAD_DST)]).reshape(-1, ECHUNK)
    b0 = b_gcn[0].reshape(1, D)
    b1 = b_gcn[1].reshape(1, D)
    bout = b_out.reshape(1, NUM_CLASS)

    degp = _sc_degree(dst2d).reshape(NC, DEG_PAD).T
    t = _tc_embed(x, W_embed, W_gcn[0])
    g1 = _tc_scale(t, degp)
    acc1 = _sc_aggregate(g1, src2d, dst2d).reshape(NC, ACC_PAD, D)
    g2 = _tc_layer2_prep(acc1, g1, degp, W_gcn[1], b0)
    acc2 = _sc_aggregate(g2, src2d, dst2d).reshape(NC, ACC_PAD, D)
    batch3 = batch.reshape(_GRID, 1, _RB)
    return _tc_pool_out(acc2, g2, degp, batch3, centroids, b1, W_out, bout)


# speedup vs baseline: 3.5092x; 1.0309x over previous
import jax
import jax.numpy as jnp
from jax import lax
from jax.experimental import pallas as pl
from jax.experimental.pallas import tpu as pltpu
from jax.experimental.pallas import tpu_sc as plsc

N = 10000
E = 320000
D = 128
NUM_CENTROID = 100
NUM_CLASS = 10
NUM_GRAPHS = 128

NC = 2
NS = 16
NW = NC * NS
ACC_PAD = 10240
ROWS_PER_TILE = ACC_PAD // NS
ECHUNK = 128
NCHUNK = 80
E_PAD = NW * NCHUNK * ECHUNK
DEG_PAD = 10240
DEG_PER_TILE = DEG_PAD // NS

_HIGH = jax.lax.Precision.HIGHEST


def _mesh():
    return plsc.VectorSubcoreMesh(core_axis_name="c", subcore_axis_name="s")


def _deg_body(dst_hbm, deg_hbm, acc_sh, dstall, ones_v, zbuf):
    c = lax.axis_index("c")
    s = lax.axis_index("s")
    w = c * NS + s
    one16 = jnp.full((16,), 1.0, dtype=jnp.float32)
    zero16 = jnp.zeros((16,), dtype=jnp.float32)

    def fill_ones(k, _):
        ones_v[pl.ds(k * 16, 16)] = one16
        return 0

    lax.fori_loop(0, ECHUNK // 16, fill_ones, 0)

    def fill_zero(k, _):
        zbuf[pl.ds(k * 16, 16)] = zero16
        return 0

    lax.fori_loop(0, DEG_PER_TILE // 16, fill_zero, 0)
    pltpu.sync_copy(zbuf, acc_sh.at[pl.ds(s * DEG_PER_TILE, DEG_PER_TILE)])
    pltpu.sync_copy(dst_hbm.at[pl.ds(w * NCHUNK, NCHUNK)], dstall)
    plsc.subcore_barrier()

    def step(it, _):
        pltpu.sync_copy(ones_v, acc_sh.at[dstall.at[it]], add=True)
        return 0

    lax.fori_loop(0, jnp.minimum(NCHUNK, E // ECHUNK - w * NCHUNK), step, 0)
    plsc.subcore_barrier()
    pltpu.sync_copy(
        acc_sh.at[pl.ds(s * DEG_PER_TILE, DEG_PER_TILE)],
        deg_hbm.at[pl.ds(c * DEG_PAD + s * DEG_PER_TILE, DEG_PER_TILE)],
    )


def _sc_degree(dst2d):
    kern = pl.kernel(
        _deg_body,
        out_type=jax.ShapeDtypeStruct((NC * DEG_PAD,), jnp.float32),
        mesh=_mesh(),
        scratch_types=[
            pltpu.VMEM_SHARED((DEG_PAD,), jnp.float32),
            pltpu.VMEM((NCHUNK, ECHUNK), jnp.int32),
            pltpu.VMEM((ECHUNK,), jnp.float32),
            pltpu.VMEM((DEG_PER_TILE,), jnp.float32),
        ],
    )
    return kern(dst2d)


_PHCH = NCHUNK // 2


def _agg_body(g_hbm, src_hbm, dst_hbm, out_hbm, acc_sh, srcall, dstall,
              rows_a, rows_b, sem_a, sem_b):
    c = lax.axis_index("c")
    s = lax.axis_index("s")
    w = c * NS + s
    nch = jnp.minimum(NCHUNK, E // ECHUNK - w * NCHUNK)
    zero16 = jnp.zeros((16,), dtype=jnp.float32)

    def zrow(r, _):
        for j in range(D // 16):
            rows_a[r, pl.ds(j * 16, 16)] = zero16
        return 0

    lax.fori_loop(0, ECHUNK, zrow, 0)
    for rr in range(ROWS_PER_TILE // ECHUNK):
        pltpu.sync_copy(rows_a,
                        acc_sh.at[pl.ds(s * ROWS_PER_TILE + rr * ECHUNK, ECHUNK)])
    plsc.subcore_barrier()

    for ph in range(NCHUNK // _PHCH):
        cnt = jnp.clip(nch - ph * _PHCH, 0, _PHCH)

        def step(k, _):
            it0 = 2 * k
            it1 = 2 * k + 1
            pltpu.async_copy(g_hbm.at[srcall.at[it1]], rows_b, sem_b)
            pltpu.make_async_copy(g_hbm.at[srcall.at[it0]], rows_a, sem_a).wait()
            pltpu.sync_copy(rows_a, acc_sh.at[dstall.at[it0]], add=True)

            @pl.when(k < cnt // 2 - 1)
            def _():
                pltpu.async_copy(g_hbm.at[srcall.at[it1 + 1]], rows_a, sem_a)

            pltpu.make_async_copy(g_hbm.at[srcall.at[it1]], rows_b, sem_b).wait()
            pltpu.sync_copy(rows_b, acc_sh.at[dstall.at[it1]], add=True)
            return 0

        @pl.when(cnt > 0)
        def _():
            pltpu.sync_copy(src_hbm.at[pl.ds(w * NCHUNK + ph * _PHCH, _PHCH)],
                            srcall)
            pltpu.sync_copy(dst_hbm.at[pl.ds(w * NCHUNK + ph * _PHCH, _PHCH)],
                            dstall)
            pltpu.async_copy(g_hbm.at[srcall.at[0]], rows_a, sem_a)

        lax.fori_loop(0, cnt // 2, step, 0)

    plsc.subcore_barrier()
    pltpu.sync_copy(
        acc_sh.at[pl.ds(s * ROWS_PER_TILE, ROWS_PER_TILE)],
        out_hbm.at[pl.ds(c * ACC_PAD + s * ROWS_PER_TILE, ROWS_PER_TILE)],
    )


def _sc_aggregate(g, src2d, dst2d):
    kern = pl.kernel(
        _agg_body,
        out_type=jax.ShapeDtypeStruct((NC * ACC_PAD, D), jnp.float32),
        mesh=_mesh(),
        scratch_types=[
            pltpu.VMEM_SHARED((ACC_PAD, D), jnp.float32),
            pltpu.VMEM((_PHCH, ECHUNK), jnp.int32),
            pltpu.VMEM((_PHCH, ECHUNK), jnp.int32),
            pltpu.VMEM((ECHUNK, D), jnp.float32),
            pltpu.VMEM((ECHUNK, D), jnp.float32),
            pltpu.SemaphoreType.DMA,
            pltpu.SemaphoreType.DMA,
        ],
    )
    return kern(g, src2d, dst2d)


_RB = 1000
_GRID = N // _RB
_DEF = jax.lax.Precision.DEFAULT


def _dinv_col(degp_blk):
    return lax.rsqrt(degp_blk[:, 0:1] + degp_blk[:, 1:2] + 1.0)


_EB = 32768


def _split_body(ei_ref, src_ref, dst_ref):
    src_ref[...] = ei_ref[0, :].reshape(_EB // ECHUNK, ECHUNK)
    dst_ref[...] = ei_ref[1, :].reshape(_EB // ECHUNK, ECHUNK)


def _tc_split_edges(ei):
    return pl.pallas_call(
        _split_body,
        grid=(10,),
        in_specs=[pl.BlockSpec((2, _EB), lambda i: (0, i))],
        out_specs=[
            pl.BlockSpec((_EB // ECHUNK, ECHUNK), lambda i: (i, 0)),
            pl.BlockSpec((_EB // ECHUNK, ECHUNK), lambda i: (i, 0)),
        ],
        out_shape=[
            jax.ShapeDtypeStruct((E_PAD // ECHUNK, ECHUNK), jnp.int32),
            jax.ShapeDtypeStruct((E_PAD // ECHUNK, ECHUNK), jnp.int32),
        ],
    )(ei)


def _k2a_body(x_ref, we_ref, w0_ref, t_ref):
    t = lax.dot_general(x_ref[...], we_ref[...], (((1,), (0,)), ((), ())),
                        precision=_DEF, preferred_element_type=jnp.float32)
    t_ref[...] = lax.dot_general(t, w0_ref[...], (((1,), (0,)), ((), ())),
                                 precision=_DEF,
                                 preferred_element_type=jnp.float32)


def _tc_embed(x, W_embed, W0):
    return pl.pallas_call(
        _k2a_body,
        grid=(_GRID,),
        in_specs=[
            pl.BlockSpec((_RB, D), lambda i: (i, 0)),
            pl.BlockSpec((D, D), lambda i: (0, 0)),
            pl.BlockSpec((D, D), lambda i: (0, 0)),
        ],
        out_specs=pl.BlockSpec((_RB, D), lambda i: (i, 0)),
        out_shape=jax.ShapeDtypeStruct((N, D), jnp.float32),
    )(x, W_embed, W0)


def _k2b_body(t_ref, degp_ref, g1_ref):
    g1_ref[...] = _dinv_col(degp_ref[...]) * t_ref[...]


def _tc_scale(t, degp):
    return pl.pallas_call(
        _k2b_body,
        grid=(_GRID,),
        in_specs=[
            pl.BlockSpec((_RB, D), lambda i: (i, 0)),
            pl.BlockSpec((_RB, NC), lambda i: (i, 0)),
        ],
        out_specs=pl.BlockSpec((_RB, D), lambda i: (i, 0)),
        out_shape=jax.ShapeDtypeStruct((N, D), jnp.float32),
    )(t, degp)


def _k4_body(acc_ref, g1_ref, degp_ref, w1_ref, b0_ref, g2_ref):
    dv = _dinv_col(degp_ref[...])
    a = acc_ref[0] + acc_ref[1] + g1_ref[...]
    h1 = jnp.maximum(dv * a + b0_ref[...], 0.0)
    t = lax.dot_general(h1, w1_ref[...], (((1,), (0,)), ((), ())),
                        precision=_DEF, preferred_element_type=jnp.float32)
    g2_ref[...] = dv * t


def _tc_layer2_prep(acc1, g1, degp, W1, b0):
    return pl.pallas_call(
        _k4_body,
        grid=(_GRID,),
        in_specs=[
            pl.BlockSpec((NC, _RB, D), lambda i: (0, i, 0)),
            pl.BlockSpec((_RB, D), lambda i: (i, 0)),
            pl.BlockSpec((_RB, NC), lambda i: (i, 0)),
            pl.BlockSpec((D, D), lambda i: (0, 0)),
            pl.BlockSpec((1, D), lambda i: (0, 0)),
        ],
        out_specs=pl.BlockSpec((_RB, D), lambda i: (i, 0)),
        out_shape=jax.ShapeDtypeStruct((N, D), jnp.float32),
    )(acc1, g1, degp, W1, b0)


def _k6_body(acc_ref, g2_ref, degp_ref, batch_ref, cent_ref, b1_ref,
             wout_ref, bout_ref, out_ref, pacc):
    i = pl.program_id(0)

    @pl.when(i == 0)
    def _init():
        pacc[...] = jnp.zeros_like(pacc)

    dv = _dinv_col(degp_ref[...])
    a = acc_ref[0] + acc_ref[1] + g2_ref[...]
    h2 = jnp.maximum(dv * a + b1_ref[...], 0.0)

    cent = cent_ref[...]
    csq = jnp.sum(cent * cent, axis=1)
    hc = lax.dot_general(h2, cent, (((1,), (1,)), ((), ())),
                         precision=_DEF, preferred_element_type=jnp.float32)
    sq = jnp.sum(h2 * h2, axis=1, keepdims=True) + csq[None, :] - 2.0 * hc
    dist = jnp.sqrt(jnp.maximum(sq, 1e-8))
    dist1 = jnp.concatenate(
        [dist, jnp.ones((dist.shape[0], 1), jnp.float32)], axis=1)

    gids = lax.broadcasted_iota(jnp.int32, (NUM_GRAPHS, _RB), 0)
    bt = batch_ref[...].reshape(1, _RB)
    oht = (bt == gids).astype(jnp.float32)
    pacc[...] += lax.dot_general(oht, dist1, (((1,), (0,)), ((), ())),
                                 precision=_HIGH,
                                 preferred_element_type=jnp.float32)

    @pl.when(i == _GRID - 1)
    def _final():
        p = pacc[...]
        pooled = p[:, :NUM_CENTROID] / jnp.maximum(p[:, NUM_CENTROID:], 1.0)
        out_ref[...] = lax.dot_general(
            pooled, wout_ref[...], (((1,), (0,)), ((), ())),
            precision=_HIGH, preferred_element_type=jnp.float32) + bout_ref[...]


def _tc_pool_out(acc2, g2, degp, batch, centroids, b1, W_out, b_out):
    return pl.pallas_call(
        _k6_body,
        grid=(_GRID,),
        in_specs=[
            pl.BlockSpec((NC, _RB, D), lambda i: (0, i, 0)),
            pl.BlockSpec((_RB, D), lambda i: (i, 0)),
            pl.BlockSpec((_RB, NC), lambda i: (i, 0)),
            pl.BlockSpec((1, 1, _RB), lambda i: (i, 0, 0)),
            pl.BlockSpec((NUM_CENTROID, D), lambda i: (0, 0)),
            pl.BlockSpec((1, D), lambda i: (0, 0)),
            pl.BlockSpec((NUM_CENTROID, NUM_CLASS), lambda i: (0, 0)),
            pl.BlockSpec((1, NUM_CLASS), lambda i: (0, 0)),
        ],
        out_specs=pl.BlockSpec((NUM_GRAPHS, NUM_CLASS), lambda i: (0, 0)),
        out_shape=jax.ShapeDtypeStruct((NUM_GRAPHS, NUM_CLASS), jnp.float32),
        scratch_shapes=[pltpu.VMEM((NUM_GRAPHS, NUM_CENTROID + 1), jnp.float32)],
    )(acc2, g2, degp, batch, centroids, b1, W_out, b_out)


def kernel(x, edge_index, batch, W_embed, W_gcn, b_gcn, centroids, W_out, b_out):
    src2d, dst2d = _tc_split_edges(edge_index)
    b0 = b_gcn[0].reshape(1, D)
    b1 = b_gcn[1].reshape(1, D)
    bout = b_out.reshape(1, NUM_CLASS)

    degp = _sc_degree(dst2d).reshape(NC, DEG_PAD).T
    t = _tc_embed(x, W_embed, W_gcn[0])
    g1 = _tc_scale(t, degp)
    acc1 = _sc_aggregate(g1, src2d, dst2d).reshape(NC, ACC_PAD, D)
    g2 = _tc_layer2_prep(acc1, g1, degp, W_gcn[1], b0)
    acc2 = _sc_aggregate(g2, src2d, dst2d).reshape(NC, ACC_PAD, D)
    batch3 = batch.reshape(_GRID, 1, _RB)
    return _tc_pool_out(acc2, g2, degp, batch3, centroids, b1, W_out, bout)


# speedup vs baseline: 3.6658x; 1.0446x over previous
import jax
import jax.numpy as jnp
from jax import lax
from jax.experimental import pallas as pl
from jax.experimental.pallas import tpu as pltpu
from jax.experimental.pallas import tpu_sc as plsc

N = 10000
E = 320000
D = 128
NUM_CENTROID = 100
NUM_CLASS = 10
NUM_GRAPHS = 128

NC = 2
NS = 16
NW = NC * NS
ACC_PAD = 10240
ROWS_PER_TILE = ACC_PAD // NS
ECHUNK = 128
NCHUNK = 80
E_PAD = NW * NCHUNK * ECHUNK
DEG_PAD = 10240
DEG_PER_TILE = DEG_PAD // NS

_HIGH = jax.lax.Precision.HIGHEST


def _mesh():
    return plsc.VectorSubcoreMesh(core_axis_name="c", subcore_axis_name="s")


def _deg_body(dst_hbm, deg_hbm, acc_sh, dstall, ones_v, zbuf):
    c = lax.axis_index("c")
    s = lax.axis_index("s")
    w = c * NS + s
    one16 = jnp.full((16,), 1.0, dtype=jnp.float32)
    zero16 = jnp.zeros((16,), dtype=jnp.float32)

    def fill_ones(k, _):
        ones_v[pl.ds(k * 16, 16)] = one16
        return 0

    lax.fori_loop(0, ECHUNK // 16, fill_ones, 0)

    def fill_zero(k, _):
        zbuf[pl.ds(k * 16, 16)] = zero16
        return 0

    lax.fori_loop(0, DEG_PER_TILE // 16, fill_zero, 0)
    pltpu.sync_copy(zbuf, acc_sh.at[pl.ds(s * DEG_PER_TILE, DEG_PER_TILE)])
    pltpu.sync_copy(dst_hbm.at[pl.ds(w * NCHUNK, NCHUNK)], dstall)
    plsc.subcore_barrier()

    def step(it, _):
        pltpu.sync_copy(ones_v, acc_sh.at[dstall.at[it]], add=True)
        return 0

    lax.fori_loop(0, jnp.minimum(NCHUNK, E // ECHUNK - w * NCHUNK), step, 0)
    plsc.subcore_barrier()
    pltpu.sync_copy(
        acc_sh.at[pl.ds(s * DEG_PER_TILE, DEG_PER_TILE)],
        deg_hbm.at[pl.ds(c * DEG_PAD + s * DEG_PER_TILE, DEG_PER_TILE)],
    )


def _sc_degree(dst2d):
    kern = pl.kernel(
        _deg_body,
        out_type=jax.ShapeDtypeStruct((NC * DEG_PAD,), jnp.float32),
        mesh=_mesh(),
        scratch_types=[
            pltpu.VMEM_SHARED((DEG_PAD,), jnp.float32),
            pltpu.VMEM((NCHUNK, ECHUNK), jnp.int32),
            pltpu.VMEM((ECHUNK,), jnp.float32),
            pltpu.VMEM((DEG_PER_TILE,), jnp.float32),
        ],
    )
    return kern(dst2d)


_PHCH = NCHUNK // 2


def _agg_body(g_hbm, src_hbm, dst_hbm, out_hbm, acc_sh, srcall, dstall,
              rows_a, rows_b, sem_a, sem_b):
    c = lax.axis_index("c")
    s = lax.axis_index("s")
    w = c * NS + s
    nch = jnp.minimum(NCHUNK, E // ECHUNK - w * NCHUNK)
    zero16 = jnp.zeros((16,), dtype=jnp.float32)

    def zrow(r, _):
        for j in range(D // 16):
            rows_a[r, pl.ds(j * 16, 16)] = zero16
        return 0

    lax.fori_loop(0, ECHUNK, zrow, 0)
    for rr in range(ROWS_PER_TILE // ECHUNK):
        pltpu.sync_copy(rows_a,
                        acc_sh.at[pl.ds(s * ROWS_PER_TILE + rr * ECHUNK, ECHUNK)])
    plsc.subcore_barrier()

    for ph in range(NCHUNK // _PHCH):
        cnt = jnp.clip(nch - ph * _PHCH, 0, _PHCH)

        def step(k, _):
            it0 = 2 * k
            it1 = 2 * k + 1
            pltpu.async_copy(g_hbm.at[srcall.at[it1]], rows_b, sem_b)
            pltpu.make_async_copy(g_hbm.at[srcall.at[it0]], rows_a, sem_a).wait()
            pltpu.sync_copy(rows_a, acc_sh.at[dstall.at[it0]], add=True)

            @pl.when(k < cnt // 2 - 1)
            def _():
                pltpu.async_copy(g_hbm.at[srcall.at[it1 + 1]], rows_a, sem_a)

            pltpu.make_async_copy(g_hbm.at[srcall.at[it1]], rows_b, sem_b).wait()
            pltpu.sync_copy(rows_b, acc_sh.at[dstall.at[it1]], add=True)
            return 0

        @pl.when(cnt > 0)
        def _():
            pltpu.sync_copy(src_hbm.at[pl.ds(w * NCHUNK + ph * _PHCH, _PHCH)],
                            srcall)
            pltpu.sync_copy(dst_hbm.at[pl.ds(w * NCHUNK + ph * _PHCH, _PHCH)],
                            dstall)
            pltpu.async_copy(g_hbm.at[srcall.at[0]], rows_a, sem_a)

        lax.fori_loop(0, cnt // 2, step, 0)

    plsc.subcore_barrier()
    pltpu.sync_copy(
        acc_sh.at[pl.ds(s * ROWS_PER_TILE, ROWS_PER_TILE)],
        out_hbm.at[pl.ds(c * ACC_PAD + s * ROWS_PER_TILE, ROWS_PER_TILE)],
    )


def _sc_aggregate(g, src2d, dst2d):
    kern = pl.kernel(
        _agg_body,
        out_type=jax.ShapeDtypeStruct((NC * ACC_PAD, D), jnp.float32),
        mesh=_mesh(),
        scratch_types=[
            pltpu.VMEM_SHARED((ACC_PAD, D), jnp.float32),
            pltpu.VMEM((_PHCH, ECHUNK), jnp.int32),
            pltpu.VMEM((_PHCH, ECHUNK), jnp.int32),
            pltpu.VMEM((ECHUNK, D), jnp.float32),
            pltpu.VMEM((ECHUNK, D), jnp.float32),
            pltpu.SemaphoreType.DMA,
            pltpu.SemaphoreType.DMA,
        ],
    )
    return kern(g, src2d, dst2d)


_RB = 2000
_GRID = N // _RB
_DEF = jax.lax.Precision.DEFAULT


def _dinv_col(degp_blk):
    return lax.rsqrt(degp_blk[:, 0:1] + degp_blk[:, 1:2] + 1.0)


_EB = 32768


def _split_body(ei_ref, src_ref, dst_ref):
    src_ref[...] = ei_ref[0, :].reshape(_EB // ECHUNK, ECHUNK)
    dst_ref[...] = ei_ref[1, :].reshape(_EB // ECHUNK, ECHUNK)


def _tc_split_edges(ei):
    return pl.pallas_call(
        _split_body,
        grid=(10,),
        in_specs=[pl.BlockSpec((2, _EB), lambda i: (0, i))],
        out_specs=[
            pl.BlockSpec((_EB // ECHUNK, ECHUNK), lambda i: (i, 0)),
            pl.BlockSpec((_EB // ECHUNK, ECHUNK), lambda i: (i, 0)),
        ],
        out_shape=[
            jax.ShapeDtypeStruct((E_PAD // ECHUNK, ECHUNK), jnp.int32),
            jax.ShapeDtypeStruct((E_PAD // ECHUNK, ECHUNK), jnp.int32),
        ],
    )(ei)


def _k2a_body(x_ref, we_ref, w0_ref, t_ref):
    t = lax.dot_general(x_ref[...], we_ref[...], (((1,), (0,)), ((), ())),
                        precision=_DEF, preferred_element_type=jnp.float32)
    t_ref[...] = lax.dot_general(t, w0_ref[...], (((1,), (0,)), ((), ())),
                                 precision=_DEF,
                                 preferred_element_type=jnp.float32)


def _tc_embed(x, W_embed, W0):
    return pl.pallas_call(
        _k2a_body,
        grid=(_GRID,),
        in_specs=[
            pl.BlockSpec((_RB, D), lambda i: (i, 0)),
            pl.BlockSpec((D, D), lambda i: (0, 0)),
            pl.BlockSpec((D, D), lambda i: (0, 0)),
        ],
        out_specs=pl.BlockSpec((_RB, D), lambda i: (i, 0)),
        out_shape=jax.ShapeDtypeStruct((N, D), jnp.float32),
    )(x, W_embed, W0)


def _k2b_body(t_ref, degp_ref, g1_ref):
    g1_ref[...] = _dinv_col(degp_ref[...]) * t_ref[...]


def _tc_scale(t, degp):
    return pl.pallas_call(
        _k2b_body,
        grid=(_GRID,),
        in_specs=[
            pl.BlockSpec((_RB, D), lambda i: (i, 0)),
            pl.BlockSpec((_RB, NC), lambda i: (i, 0)),
        ],
        out_specs=pl.BlockSpec((_RB, D), lambda i: (i, 0)),
        out_shape=jax.ShapeDtypeStruct((N, D), jnp.float32),
    )(t, degp)


def _k4_body(acc_ref, g1_ref, degp_ref, w1_ref, b0_ref, g2_ref):
    dv = _dinv_col(degp_ref[...])
    a = acc_ref[0] + acc_ref[1] + g1_ref[...]
    h1 = jnp.maximum(dv * a + b0_ref[...], 0.0)
    t = lax.dot_general(h1, w1_ref[...], (((1,), (0,)), ((), ())),
                        precision=_DEF, preferred_element_type=jnp.float32)
    g2_ref[...] = dv * t


def _tc_layer2_prep(acc1, g1, degp, W1, b0):
    return pl.pallas_call(
        _k4_body,
        grid=(_GRID,),
        in_specs=[
            pl.BlockSpec((NC, _RB, D), lambda i: (0, i, 0)),
            pl.BlockSpec((_RB, D), lambda i: (i, 0)),
            pl.BlockSpec((_RB, NC), lambda i: (i, 0)),
            pl.BlockSpec((D, D), lambda i: (0, 0)),
            pl.BlockSpec((1, D), lambda i: (0, 0)),
        ],
        out_specs=pl.BlockSpec((_RB, D), lambda i: (i, 0)),
        out_shape=jax.ShapeDtypeStruct((N, D), jnp.float32),
    )(acc1, g1, degp, W1, b0)


def _k6_body(acc_ref, g2_ref, degp_ref, batch_ref, cent_ref, b1_ref,
             wout_ref, bout_ref, out_ref, pacc):
    i = pl.program_id(0)

    @pl.when(i == 0)
    def _init():
        pacc[...] = jnp.zeros_like(pacc)

    dv = _dinv_col(degp_ref[...])
    a = acc_ref[0] + acc_ref[1] + g2_ref[...]
    h2 = jnp.maximum(dv * a + b1_ref[...], 0.0)

    cent = cent_ref[...]
    csq = jnp.sum(cent * cent, axis=1)
    hc = lax.dot_general(h2, cent, (((1,), (1,)), ((), ())),
                         precision=_DEF, preferred_element_type=jnp.float32)
    sq = jnp.sum(h2 * h2, axis=1, keepdims=True) + csq[None, :] - 2.0 * hc
    dist = jnp.sqrt(jnp.maximum(sq, 1e-8))
    dist1 = jnp.concatenate(
        [dist, jnp.ones((dist.shape[0], 1), jnp.float32)], axis=1)

    gids = lax.broadcasted_iota(jnp.int32, (NUM_GRAPHS, _RB), 0)
    bt = batch_ref[...].reshape(1, _RB)
    oht = (bt == gids).astype(jnp.float32)
    pacc[...] += lax.dot_general(oht, dist1, (((1,), (0,)), ((), ())),
                                 precision=_DEF,
                                 preferred_element_type=jnp.float32)

    @pl.when(i == _GRID - 1)
    def _final():
        p = pacc[...]
        pooled = p[:, :NUM_CENTROID] / jnp.maximum(p[:, NUM_CENTROID:], 1.0)
        out_ref[...] = lax.dot_general(
            pooled, wout_ref[...], (((1,), (0,)), ((), ())),
            precision=_HIGH, preferred_element_type=jnp.float32) + bout_ref[...]


def _tc_pool_out(acc2, g2, degp, batch, centroids, b1, W_out, b_out):
    return pl.pallas_call(
        _k6_body,
        grid=(_GRID,),
        in_specs=[
            pl.BlockSpec((NC, _RB, D), lambda i: (0, i, 0)),
            pl.BlockSpec((_RB, D), lambda i: (i, 0)),
            pl.BlockSpec((_RB, NC), lambda i: (i, 0)),
            pl.BlockSpec((1, 1, _RB), lambda i: (i, 0, 0)),
            pl.BlockSpec((NUM_CENTROID, D), lambda i: (0, 0)),
            pl.BlockSpec((1, D), lambda i: (0, 0)),
            pl.BlockSpec((NUM_CENTROID, NUM_CLASS), lambda i: (0, 0)),
            pl.BlockSpec((1, NUM_CLASS), lambda i: (0, 0)),
        ],
        out_specs=pl.BlockSpec((NUM_GRAPHS, NUM_CLASS), lambda i: (0, 0)),
        out_shape=jax.ShapeDtypeStruct((NUM_GRAPHS, NUM_CLASS), jnp.float32),
        scratch_shapes=[pltpu.VMEM((NUM_GRAPHS, NUM_CENTROID + 1), jnp.float32)],
    )(acc2, g2, degp, batch, centroids, b1, W_out, b_out)


def kernel(x, edge_index, batch, W_embed, W_gcn, b_gcn, centroids, W_out, b_out):
    src2d, dst2d = _tc_split_edges(edge_index)
    b0 = b_gcn[0].reshape(1, D)
    b1 = b_gcn[1].reshape(1, D)
    bout = b_out.reshape(1, NUM_CLASS)

    degp = _sc_degree(dst2d).reshape(NC, DEG_PAD).T
    t = _tc_embed(x, W_embed, W_gcn[0])
    g1 = _tc_scale(t, degp)
    acc1 = _sc_aggregate(g1, src2d, dst2d).reshape(NC, ACC_PAD, D)
    g2 = _tc_layer2_prep(acc1, g1, degp, W_gcn[1], b0)
    acc2 = _sc_aggregate(g2, src2d, dst2d).reshape(NC, ACC_PAD, D)
    batch3 = batch.reshape(_GRID, 1, _RB)
    return _tc_pool_out(acc2, g2, degp, batch3, centroids, b1, W_out, bout)
